# baseline (speedup 1.0000x reference)
import jax
import jax.numpy as jnp
from jax import lax
from jax.experimental import pallas as pl

HIDDEN = 2048
NUM_EXPERTS = 16
TOP_K = 2
BLOCK = 1024


def _body(x_ref, wt_ref, logits_ref, idx_ref, w_ref):
    logits = jnp.dot(x_ref[...], wt_ref[...], preferred_element_type=jnp.float32)
    b = logits.shape[0]
    iota = lax.broadcasted_iota(jnp.int32, (b, NUM_EXPERTS), 1)
    m1 = jnp.max(logits, axis=1, keepdims=True)
    idx1 = jnp.min(jnp.where(logits == m1, iota, NUM_EXPERTS), axis=1, keepdims=True)
    masked = jnp.where(iota == idx1, -jnp.inf, logits)
    m2 = jnp.max(masked, axis=1, keepdims=True)
    idx2 = jnp.min(jnp.where(masked == m2, iota, NUM_EXPERTS), axis=1, keepdims=True)
    e = jnp.exp(m2 - m1)
    w1 = 1.0 / (1.0 + e)
    w2 = 1.0 - w1
    logits_ref[...] = logits
    col = lax.broadcasted_iota(jnp.int32, (b, TOP_K), 1)
    idx_ref[...] = jnp.where(col == 0, idx1, idx2)
    w_ref[...] = jnp.where(col == 0, w1, w2)


def kernel(hidden_states, W):
    b, s, h = hidden_states.shape
    x = hidden_states.reshape(-1, h)
    n = x.shape[0]
    wt = W.T
    grid = (n // BLOCK,)
    out = pl.pallas_call(
        _body,
        grid=grid,
        in_specs=[
            pl.BlockSpec((BLOCK, h), lambda i: (i, 0)),
            pl.BlockSpec((h, NUM_EXPERTS), lambda i: (0, 0)),
        ],
        out_specs=[
            pl.BlockSpec((BLOCK, NUM_EXPERTS), lambda i: (i, 0)),
            pl.BlockSpec((BLOCK, TOP_K), lambda i: (i, 0)),
            pl.BlockSpec((BLOCK, TOP_K), lambda i: (i, 0)),
        ],
        out_shape=[
            jax.ShapeDtypeStruct((n, NUM_EXPERTS), jnp.float32),
            jax.ShapeDtypeStruct((n, TOP_K), jnp.int32),
            jax.ShapeDtypeStruct((n, TOP_K), jnp.float32),
        ],
    )(x, wt)
    return tuple(out)
